# baseline (speedup 1.0000x reference)
import dataclasses
import functools

import jax
import jax.numpy as jnp
from jax import lax
from jax.experimental import pallas as pl
from jax.experimental.pallas import tpu as pltpu
from jax.experimental.pallas import tpu_sc as plsc

N = 10000
E = 320000
CIN = 128
HID = 128
NCLS = 40

NC = 2
NS = 16
NW = NC * NS
CHUNK = 128
EP = 327680
EPW = EP // NW
NCH = EPW // CHUNK
NP = 10240
RPT = NP // NS
DUMP = 10200

_MESH = plsc.VectorSubcoreMesh(core_axis_name="c", subcore_axis_name="s")
_F32 = jnp.float32

_CP = pltpu.CompilerParams()
if "needs_layout_passes" in pltpu.CompilerParams.__dataclass_fields__:
    _CP = dataclasses.replace(_CP, needs_layout_passes=False)


def _deg_kernel(col_hbm):
    out = jax.ShapeDtypeStruct((NW * NP,), _F32)

    @functools.partial(
        pl.kernel,
        out_type=out,
        mesh=_MESH,
        compiler_params=_CP,
        scratch_types=[
            pltpu.VMEM((NP,), _F32),
            pltpu.VMEM((NCH, CHUNK), jnp.int32),
        ],
    )
    def deg(col_ref, out_ref, hist, colv):
        sid = lax.axis_index("s")
        cid = lax.axis_index("c")
        wid = sid * NC + cid
        pltpu.sync_copy(col_ref.at[wid], colv)

        @pl.loop(0, NP // 16)
        def _(i):
            hist[pl.ds(i * 16, 16)] = jnp.zeros((16,), _F32)

        @pl.loop(0, NCH)
        def _(j):
            @pl.loop(0, CHUNK // 16)
            def _(c):
                idx = colv[j, pl.ds(c * 16, 16)]
                plsc.addupdate_scatter(hist, [idx], jnp.ones((16,), _F32))

        pltpu.sync_copy(hist, out_ref.at[pl.ds(wid * NP, NP)])

    return deg(col_hbm)


GLEN = 32
NG0 = 5


def _agg_kernel(y_hbm, row_hbm, col_hbm):
    out = jax.ShapeDtypeStruct((NP, HID), _F32)

    @functools.partial(
        pl.kernel,
        out_type=out,
        mesh=_MESH,
        scratch_types=[
            pltpu.VMEM_SHARED((NP, HID), _F32),
            pltpu.VMEM((GLEN, CHUNK), jnp.int32),
            pltpu.VMEM((GLEN, CHUNK), jnp.int32),
            pltpu.VMEM((CHUNK, HID), _F32),
            pltpu.VMEM((CHUNK, HID), _F32),
            pltpu.SemaphoreType.DMA,
            pltpu.SemaphoreType.DMA,
            pltpu.SemaphoreType.DMA,
            pltpu.SemaphoreType.DMA,
        ],
    )
    def agg(y_ref, row_ref, col_ref, out_ref,
            acc, rowv, colv, gbuf0, gbuf1, sem0, sem1, ssem0, ssem1):
        sid = lax.axis_index("s")
        cid = lax.axis_index("c")
        active = 1 - cid
        ngrp = NG0 * active
        cbase = sid * (NG0 * GLEN)

        @pl.loop(0, CHUNK * HID // 16)
        def _(i):
            r = i // (HID // 16)
            c = (i % (HID // 16)) * 16
            gbuf0[r, pl.ds(c, 16)] = jnp.zeros((16,), _F32)

        @pl.loop(0, (RPT // CHUNK) * active)
        def _(k):
            pltpu.sync_copy(gbuf0, acc.at[pl.ds(sid * RPT + k * CHUNK, CHUNK)])

        plsc.subcore_barrier()

        bufs = (gbuf0, gbuf1)
        sems = (sem0, sem1)
        ssems = (ssem0, ssem1)

        @pl.loop(0, ngrp)
        def _(g):
            base = cbase + g * GLEN
            pltpu.sync_copy(row_ref.at[pl.ds(base, GLEN)], rowv)
            pltpu.sync_copy(col_ref.at[pl.ds(base, GLEN)], colv)
            gd = [None, None]
            sd = [None, None]
            gd[0] = pltpu.async_copy(y_ref.at[rowv.at[0]], bufs[0], sems[0])
            for t in range(GLEN):
                cur = t % 2
                gd[cur].wait()
                sd[cur] = pltpu.async_copy(bufs[cur], acc.at[colv.at[t]],
                                           ssems[cur], add=True)
                if t + 1 < GLEN:
                    if sd[1 - cur] is not None:
                        sd[1 - cur].wait()
                    gd[1 - cur] = pltpu.async_copy(y_ref.at[rowv.at[t + 1]],
                                                   bufs[1 - cur], sems[1 - cur])
            sd[1].wait()

        plsc.subcore_barrier()

        @pl.loop(0, active)
        def _(_k):
            pltpu.sync_copy(acc.at[pl.ds(sid * RPT, RPT)],
                            out_ref.at[pl.ds(sid * RPT, RPT)])

    return agg(y_hbm, row_hbm, col_hbm)


BLK = 1000


def _mm1_body(x_ref, w_ref, o_ref):
    o_ref[...] = jnp.dot(x_ref[...], w_ref[...],
                         preferred_element_type=_F32,
                         precision=lax.Precision.HIGHEST)


def _matmul1(x, w):
    return pl.pallas_call(
        _mm1_body,
        grid=(N // BLK,),
        in_specs=[
            pl.BlockSpec((BLK, CIN), lambda i: (i, 0)),
            pl.BlockSpec((CIN, HID), lambda i: (0, 0)),
        ],
        out_specs=pl.BlockSpec((BLK, HID), lambda i: (i, 0)),
        out_shape=jax.ShapeDtypeStruct((N, HID), _F32),
    )(x, w)


def _scale_body(xw_ref, deg_ref, y_ref):
    y_ref[...] = xw_ref[...] * lax.rsqrt(deg_ref[...])


def _scale(xw, deg):
    return pl.pallas_call(
        _scale_body,
        grid=(N // BLK,),
        in_specs=[
            pl.BlockSpec((BLK, HID), lambda i: (i, 0)),
            pl.BlockSpec((BLK, 1), lambda i: (i, 0)),
        ],
        out_specs=pl.BlockSpec((BLK, HID), lambda i: (i, 0)),
        out_shape=jax.ShapeDtypeStruct((N, HID), _F32),
    )(xw, deg)


def _mid_body(s0_ref, y1_ref, deg_ref, b1_ref, w2_ref, y2_ref):
    dinv = lax.rsqrt(deg_ref[...])
    h = (s0_ref[...] + y1_ref[...]) * dinv + b1_ref[...]
    h = jnp.maximum(h, 0.0)
    y2_ref[...] = jnp.dot(h, w2_ref[...],
                          preferred_element_type=_F32,
                          precision=lax.Precision.HIGHEST) * dinv


def _mid(s0, y1, deg, b1, w2p):
    return pl.pallas_call(
        _mid_body,
        grid=(N // BLK,),
        in_specs=[
            pl.BlockSpec((BLK, HID), lambda i: (i, 0)),
            pl.BlockSpec((BLK, HID), lambda i: (i, 0)),
            pl.BlockSpec((BLK, 1), lambda i: (i, 0)),
            pl.BlockSpec((1, HID), lambda i: (0, 0)),
            pl.BlockSpec((HID, HID), lambda i: (0, 0)),
        ],
        out_specs=pl.BlockSpec((BLK, HID), lambda i: (i, 0)),
        out_shape=jax.ShapeDtypeStruct((N, HID), _F32),
    )(s0, y1, deg, b1, w2p)


def _fin_body(s0_ref, y2_ref, deg_ref, b2_ref, o_ref):
    dinv = lax.rsqrt(deg_ref[...])
    t = (s0_ref[...] + y2_ref[...]) * dinv
    o_ref[...] = t[:, :NCLS] + b2_ref[...]


def _fin(s0, y2, deg, b2):
    return pl.pallas_call(
        _fin_body,
        grid=(N // BLK,),
        in_specs=[
            pl.BlockSpec((BLK, HID), lambda i: (i, 0)),
            pl.BlockSpec((BLK, HID), lambda i: (i, 0)),
            pl.BlockSpec((BLK, 1), lambda i: (i, 0)),
            pl.BlockSpec((1, NCLS), lambda i: (0, 0)),
        ],
        out_specs=pl.BlockSpec((BLK, NCLS), lambda i: (i, 0)),
        out_shape=jax.ShapeDtypeStruct((N, NCLS), _F32),
    )(s0, y2, deg, b2)


def kernel(x, edge_index, W1, b1, W2, b2):
    ei = edge_index.astype(jnp.int32)
    rowp = jnp.concatenate([ei[0], jnp.zeros((EP - E,), jnp.int32)])
    colp = jnp.concatenate([ei[1], jnp.full((EP - E,), DUMP, jnp.int32)])
    row3 = rowp.reshape(NW, NCH, CHUNK)
    col3 = colp.reshape(NW, NCH, CHUNK)
    row2 = rowp.reshape(NW * NCH, CHUNK)
    col2 = colp.reshape(NW * NCH, CHUNK)
    b1r = b1.reshape(1, HID)
    b2r = b2.reshape(1, NCLS)
    w2p = jnp.concatenate([W2, jnp.zeros((HID, HID - NCLS), _F32)], axis=1)

    hist = _deg_kernel(col3)
    xw = _matmul1(x, W1)
    deg = (hist.reshape(NW, NP).sum(axis=0)[:N] + 1.0).reshape(N, 1)
    y1 = _scale(xw, deg)
    s1 = _agg_kernel(y1, row2, col2)
    y2 = _mid(s1[:N], y1, deg, b1r, w2p)
    s2 = _agg_kernel(y2, row2, col2)
    return _fin(s2[:N], y2, deg, b2r)


# speedup vs baseline: 1.3555x; 1.3555x over previous
import dataclasses
import functools

import jax
import jax.numpy as jnp
from jax import lax
from jax.experimental import pallas as pl
from jax.experimental.pallas import tpu as pltpu
from jax.experimental.pallas import tpu_sc as plsc

N = 10000
E = 320000
CIN = 128
HID = 128
NCLS = 40

NC = 2
NS = 16
NW = NC * NS
CHUNK = 128
EP = 327680
EPW = EP // NW
NCH = EPW // CHUNK
NP = 10240
RPT = NP // NS
DUMP = 10200

_MESH = plsc.VectorSubcoreMesh(core_axis_name="c", subcore_axis_name="s")
_F32 = jnp.float32

_CP = pltpu.CompilerParams()
if "needs_layout_passes" in pltpu.CompilerParams.__dataclass_fields__:
    _CP = dataclasses.replace(_CP, needs_layout_passes=False)


def _deg_kernel(col_hbm):
    out = jax.ShapeDtypeStruct((NW * NP,), _F32)

    @functools.partial(
        pl.kernel,
        out_type=out,
        mesh=_MESH,
        compiler_params=_CP,
        scratch_types=[
            pltpu.VMEM((NP,), _F32),
            pltpu.VMEM((NCH, CHUNK), jnp.int32),
        ],
    )
    def deg(col_ref, out_ref, hist, colv):
        sid = lax.axis_index("s")
        cid = lax.axis_index("c")
        wid = sid * NC + cid
        pltpu.sync_copy(col_ref.at[wid], colv)

        @pl.loop(0, NP // 16)
        def _(i):
            hist[pl.ds(i * 16, 16)] = jnp.zeros((16,), _F32)

        @pl.loop(0, NCH)
        def _(j):
            @pl.loop(0, CHUNK // 16)
            def _(c):
                idx = colv[j, pl.ds(c * 16, 16)]
                plsc.addupdate_scatter(hist, [idx], jnp.ones((16,), _F32))

        pltpu.sync_copy(hist, out_ref.at[pl.ds(wid * NP, NP)])

    return deg(col_hbm)


GLEN = 16
NG0 = 8
NG1 = 2
C0 = NG0 * GLEN
C1 = NG1 * GLEN


def _agg_kernel(y_hbm, row_hbm, col_hbm):
    out = jax.ShapeDtypeStruct((NC * NP, HID), _F32)

    @functools.partial(
        pl.kernel,
        out_type=out,
        mesh=_MESH,
        scratch_types=[
            pltpu.VMEM_SHARED((NP, HID), _F32),
            pltpu.VMEM((GLEN, CHUNK), jnp.int32),
            pltpu.VMEM((GLEN, CHUNK), jnp.int32),
            pltpu.VMEM((CHUNK, HID), _F32),
            pltpu.VMEM((CHUNK, HID), _F32),
            pltpu.SemaphoreType.DMA,
            pltpu.SemaphoreType.DMA,
            pltpu.SemaphoreType.DMA,
            pltpu.SemaphoreType.DMA,
        ],
    )
    def agg(y_ref, row_ref, col_ref, out_ref,
            acc, rowv, colv, gbuf0, gbuf1, sem0, sem1, ssem0, ssem1):
        sid = lax.axis_index("s")
        cid = lax.axis_index("c")
        ngrp = NG0 - cid * (NG0 - NG1)
        cbase = cid * NS * C0 + sid * (C0 - cid * (C0 - C1))

        @pl.loop(0, CHUNK * HID // 16)
        def _(i):
            r = i // (HID // 16)
            c = (i % (HID // 16)) * 16
            gbuf0[r, pl.ds(c, 16)] = jnp.zeros((16,), _F32)

        @pl.loop(0, RPT // CHUNK)
        def _(k):
            pltpu.sync_copy(gbuf0, acc.at[pl.ds(sid * RPT + k * CHUNK, CHUNK)])

        plsc.subcore_barrier()

        bufs = (gbuf0, gbuf1)
        sems = (sem0, sem1)
        ssems = (ssem0, ssem1)

        @pl.loop(0, ngrp)
        def _(g):
            base = cbase + g * GLEN
            pltpu.sync_copy(row_ref.at[pl.ds(base, GLEN)], rowv)
            pltpu.sync_copy(col_ref.at[pl.ds(base, GLEN)], colv)
            gd = [None, None]
            sd = [None, None]
            gd[0] = pltpu.async_copy(y_ref.at[rowv.at[0]], bufs[0], sems[0])
            for t in range(GLEN):
                cur = t % 2
                gd[cur].wait()
                sd[cur] = pltpu.async_copy(bufs[cur], acc.at[colv.at[t]],
                                           ssems[cur], add=True)
                if t + 1 < GLEN:
                    if sd[1 - cur] is not None:
                        sd[1 - cur].wait()
                    gd[1 - cur] = pltpu.async_copy(y_ref.at[rowv.at[t + 1]],
                                                   bufs[1 - cur], sems[1 - cur])
            sd[1].wait()

        plsc.subcore_barrier()
        pltpu.sync_copy(acc.at[pl.ds(sid * RPT, RPT)],
                        out_ref.at[pl.ds(cid * NP + sid * RPT, RPT)])

    return agg(y_hbm, row_hbm, col_hbm)


BLK = 1000


def _mm1_body(x_ref, w_ref, o_ref):
    o_ref[...] = jnp.dot(x_ref[...], w_ref[...],
                         preferred_element_type=_F32,
                         precision=lax.Precision.HIGHEST)


def _matmul1(x, w):
    return pl.pallas_call(
        _mm1_body,
        grid=(N // BLK,),
        in_specs=[
            pl.BlockSpec((BLK, CIN), lambda i: (i, 0)),
            pl.BlockSpec((CIN, HID), lambda i: (0, 0)),
        ],
        out_specs=pl.BlockSpec((BLK, HID), lambda i: (i, 0)),
        out_shape=jax.ShapeDtypeStruct((N, HID), _F32),
    )(x, w)


def _scale_body(xw_ref, deg_ref, y_ref):
    y_ref[...] = xw_ref[...] * lax.rsqrt(deg_ref[...])


def _scale(xw, deg):
    return pl.pallas_call(
        _scale_body,
        grid=(N // BLK,),
        in_specs=[
            pl.BlockSpec((BLK, HID), lambda i: (i, 0)),
            pl.BlockSpec((BLK, 1), lambda i: (i, 0)),
        ],
        out_specs=pl.BlockSpec((BLK, HID), lambda i: (i, 0)),
        out_shape=jax.ShapeDtypeStruct((N, HID), _F32),
    )(xw, deg)


def _mid_body(s0_ref, s1_ref, y1_ref, deg_ref, b1_ref, w2_ref, y2_ref):
    dinv = lax.rsqrt(deg_ref[...])
    h = (s0_ref[...] + s1_ref[...] + y1_ref[...]) * dinv + b1_ref[...]
    h = jnp.maximum(h, 0.0)
    y2_ref[...] = jnp.dot(h, w2_ref[...],
                          preferred_element_type=_F32,
                          precision=lax.Precision.HIGHEST) * dinv


def _mid(s0, s1, y1, deg, b1, w2p):
    return pl.pallas_call(
        _mid_body,
        grid=(N // BLK,),
        in_specs=[
            pl.BlockSpec((BLK, HID), lambda i: (i, 0)),
            pl.BlockSpec((BLK, HID), lambda i: (i, 0)),
            pl.BlockSpec((BLK, HID), lambda i: (i, 0)),
            pl.BlockSpec((BLK, 1), lambda i: (i, 0)),
            pl.BlockSpec((1, HID), lambda i: (0, 0)),
            pl.BlockSpec((HID, HID), lambda i: (0, 0)),
        ],
        out_specs=pl.BlockSpec((BLK, HID), lambda i: (i, 0)),
        out_shape=jax.ShapeDtypeStruct((N, HID), _F32),
    )(s0, s1, y1, deg, b1, w2p)


def _fin_body(s0_ref, s1_ref, y2_ref, deg_ref, b2_ref, o_ref):
    dinv = lax.rsqrt(deg_ref[...])
    t = (s0_ref[...] + s1_ref[...] + y2_ref[...]) * dinv
    o_ref[...] = t[:, :NCLS] + b2_ref[...]


def _fin(s0, s1, y2, deg, b2):
    return pl.pallas_call(
        _fin_body,
        grid=(N // BLK,),
        in_specs=[
            pl.BlockSpec((BLK, HID), lambda i: (i, 0)),
            pl.BlockSpec((BLK, HID), lambda i: (i, 0)),
            pl.BlockSpec((BLK, HID), lambda i: (i, 0)),
            pl.BlockSpec((BLK, 1), lambda i: (i, 0)),
            pl.BlockSpec((1, NCLS), lambda i: (0, 0)),
        ],
        out_specs=pl.BlockSpec((BLK, NCLS), lambda i: (i, 0)),
        out_shape=jax.ShapeDtypeStruct((N, NCLS), _F32),
    )(s0, s1, y2, deg, b2)


def kernel(x, edge_index, W1, b1, W2, b2):
    ei = edge_index.astype(jnp.int32)
    rowp = jnp.concatenate([ei[0], jnp.zeros((EP - E,), jnp.int32)])
    colp = jnp.concatenate([ei[1], jnp.full((EP - E,), DUMP, jnp.int32)])
    row3 = rowp.reshape(NW, NCH, CHUNK)
    col3 = colp.reshape(NW, NCH, CHUNK)
    row2 = rowp.reshape(NW * NCH, CHUNK)
    col2 = colp.reshape(NW * NCH, CHUNK)
    b1r = b1.reshape(1, HID)
    b2r = b2.reshape(1, NCLS)
    w2p = jnp.concatenate([W2, jnp.zeros((HID, HID - NCLS), _F32)], axis=1)

    hist = _deg_kernel(col3)
    xw = _matmul1(x, W1)
    deg = (hist.reshape(NW, NP).sum(axis=0)[:N] + 1.0).reshape(N, 1)
    y1 = _scale(xw, deg)
    s1 = _agg_kernel(y1, row2, col2)
    y2 = _mid(s1[:N], s1[NP:NP + N], y1, deg, b1r, w2p)
    s2 = _agg_kernel(y2, row2, col2)
    return _fin(s2[:N], s2[NP:NP + N], y2, deg, b2r)


# speedup vs baseline: 1.4620x; 1.0786x over previous
import dataclasses
import functools

import jax
import jax.numpy as jnp
from jax import lax
from jax.experimental import pallas as pl
from jax.experimental.pallas import tpu as pltpu
from jax.experimental.pallas import tpu_sc as plsc

N = 10000
E = 320000
CIN = 128
HID = 128
NCLS = 40

NC = 2
NS = 16
NW = NC * NS
CHUNK = 128
EP = 327680
EPW = EP // NW
NCH = EPW // CHUNK
NP = 10240
RPT = NP // NS
DUMP = 10200

_MESH = plsc.VectorSubcoreMesh(core_axis_name="c", subcore_axis_name="s")
_F32 = jnp.float32

_CP = pltpu.CompilerParams()
if "needs_layout_passes" in pltpu.CompilerParams.__dataclass_fields__:
    _CP = dataclasses.replace(_CP, needs_layout_passes=False)


def _deg_kernel(col_hbm):
    out = jax.ShapeDtypeStruct((NW * NP,), _F32)

    @functools.partial(
        pl.kernel,
        out_type=out,
        mesh=_MESH,
        compiler_params=_CP,
        scratch_types=[
            pltpu.VMEM((NP,), _F32),
            pltpu.VMEM((NCH, CHUNK), jnp.int32),
        ],
    )
    def deg(col_ref, out_ref, hist, colv):
        sid = lax.axis_index("s")
        cid = lax.axis_index("c")
        wid = sid * NC + cid
        pltpu.sync_copy(col_ref.at[wid], colv)

        @pl.loop(0, NP // 16)
        def _(i):
            hist[pl.ds(i * 16, 16)] = jnp.zeros((16,), _F32)

        @pl.loop(0, NCH)
        def _(j):
            @pl.loop(0, CHUNK // 16)
            def _(c):
                idx = colv[j, pl.ds(c * 16, 16)]
                plsc.addupdate_scatter(hist, [idx], jnp.ones((16,), _F32))

        pltpu.sync_copy(hist, out_ref.at[pl.ds(wid * NP, NP)])

    return deg(col_hbm)


GLEN = 16
NG0 = 9
NG1 = 1
C0 = NG0 * GLEN
C1 = NG1 * GLEN


def _agg_kernel(y_hbm, row_hbm, col_hbm):
    out = jax.ShapeDtypeStruct((NC * NP, HID), _F32)

    @functools.partial(
        pl.kernel,
        out_type=out,
        mesh=_MESH,
        scratch_types=[
            pltpu.VMEM_SHARED((NP, HID), _F32),
            pltpu.VMEM((GLEN, CHUNK), jnp.int32),
            pltpu.VMEM((GLEN, CHUNK), jnp.int32),
            pltpu.VMEM((CHUNK, HID), _F32),
            pltpu.VMEM((CHUNK, HID), _F32),
            pltpu.SemaphoreType.DMA,
            pltpu.SemaphoreType.DMA,
            pltpu.SemaphoreType.DMA,
            pltpu.SemaphoreType.DMA,
        ],
    )
    def agg(y_ref, row_ref, col_ref, out_ref,
            acc, rowv, colv, gbuf0, gbuf1, sem0, sem1, ssem0, ssem1):
        sid = lax.axis_index("s")
        cid = lax.axis_index("c")
        ngrp = NG0 - cid * (NG0 - NG1)
        cbase = cid * NS * C0 + sid * (C0 - cid * (C0 - C1))

        @pl.loop(0, CHUNK * HID // 16)
        def _(i):
            r = i // (HID // 16)
            c = (i % (HID // 16)) * 16
            gbuf0[r, pl.ds(c, 16)] = jnp.zeros((16,), _F32)

        @pl.loop(0, RPT // CHUNK)
        def _(k):
            pltpu.sync_copy(gbuf0, acc.at[pl.ds(sid * RPT + k * CHUNK, CHUNK)])

        plsc.subcore_barrier()

        bufs = (gbuf0, gbuf1)
        sems = (sem0, sem1)
        ssems = (ssem0, ssem1)

        @pl.loop(0, ngrp)
        def _(g):
            base = cbase + g * GLEN
            pltpu.sync_copy(row_ref.at[pl.ds(base, GLEN)], rowv)
            pltpu.sync_copy(col_ref.at[pl.ds(base, GLEN)], colv)
            gd = [None, None]
            sd = [None, None]
            gd[0] = pltpu.async_copy(y_ref.at[rowv.at[0]], bufs[0], sems[0])
            for t in range(GLEN):
                cur = t % 2
                gd[cur].wait()
                sd[cur] = pltpu.async_copy(bufs[cur], acc.at[colv.at[t]],
                                           ssems[cur], add=True)
                if t + 1 < GLEN:
                    if sd[1 - cur] is not None:
                        sd[1 - cur].wait()
                    gd[1 - cur] = pltpu.async_copy(y_ref.at[rowv.at[t + 1]],
                                                   bufs[1 - cur], sems[1 - cur])
            sd[1].wait()

        plsc.subcore_barrier()
        pltpu.sync_copy(acc.at[pl.ds(sid * RPT, RPT)],
                        out_ref.at[pl.ds(cid * NP + sid * RPT, RPT)])

    return agg(y_hbm, row_hbm, col_hbm)


BLK = 1000


def _mm1_body(x_ref, w_ref, o_ref):
    o_ref[...] = jnp.dot(x_ref[...], w_ref[...],
                         preferred_element_type=_F32,
                         precision=lax.Precision.HIGHEST)


def _matmul1(x, w):
    return pl.pallas_call(
        _mm1_body,
        grid=(N // BLK,),
        in_specs=[
            pl.BlockSpec((BLK, CIN), lambda i: (i, 0)),
            pl.BlockSpec((CIN, HID), lambda i: (0, 0)),
        ],
        out_specs=pl.BlockSpec((BLK, HID), lambda i: (i, 0)),
        out_shape=jax.ShapeDtypeStruct((N, HID), _F32),
    )(x, w)


def _scale_body(xw_ref, deg_ref, y_ref):
    y_ref[...] = xw_ref[...] * lax.rsqrt(deg_ref[...])


def _scale(xw, deg):
    return pl.pallas_call(
        _scale_body,
        grid=(N // BLK,),
        in_specs=[
            pl.BlockSpec((BLK, HID), lambda i: (i, 0)),
            pl.BlockSpec((BLK, 1), lambda i: (i, 0)),
        ],
        out_specs=pl.BlockSpec((BLK, HID), lambda i: (i, 0)),
        out_shape=jax.ShapeDtypeStruct((N, HID), _F32),
    )(xw, deg)


def _mid_body(s0_ref, s1_ref, y1_ref, deg_ref, b1_ref, w2_ref, y2_ref):
    dinv = lax.rsqrt(deg_ref[...])
    h = (s0_ref[...] + s1_ref[...] + y1_ref[...]) * dinv + b1_ref[...]
    h = jnp.maximum(h, 0.0)
    y2_ref[...] = jnp.dot(h, w2_ref[...],
                          preferred_element_type=_F32,
                          precision=lax.Precision.HIGHEST) * dinv


def _mid(s0, s1, y1, deg, b1, w2p):
    return pl.pallas_call(
        _mid_body,
        grid=(N // BLK,),
        in_specs=[
            pl.BlockSpec((BLK, HID), lambda i: (i, 0)),
            pl.BlockSpec((BLK, HID), lambda i: (i, 0)),
            pl.BlockSpec((BLK, HID), lambda i: (i, 0)),
            pl.BlockSpec((BLK, 1), lambda i: (i, 0)),
            pl.BlockSpec((1, HID), lambda i: (0, 0)),
            pl.BlockSpec((HID, HID), lambda i: (0, 0)),
        ],
        out_specs=pl.BlockSpec((BLK, HID), lambda i: (i, 0)),
        out_shape=jax.ShapeDtypeStruct((N, HID), _F32),
    )(s0, s1, y1, deg, b1, w2p)


def _fin_body(s0_ref, s1_ref, y2_ref, deg_ref, b2_ref, o_ref):
    dinv = lax.rsqrt(deg_ref[...])
    t = (s0_ref[...] + s1_ref[...] + y2_ref[...]) * dinv
    o_ref[...] = t[:, :NCLS] + b2_ref[...]


def _fin(s0, s1, y2, deg, b2):
    return pl.pallas_call(
        _fin_body,
        grid=(N // BLK,),
        in_specs=[
            pl.BlockSpec((BLK, HID), lambda i: (i, 0)),
            pl.BlockSpec((BLK, HID), lambda i: (i, 0)),
            pl.BlockSpec((BLK, HID), lambda i: (i, 0)),
            pl.BlockSpec((BLK, 1), lambda i: (i, 0)),
            pl.BlockSpec((1, NCLS), lambda i: (0, 0)),
        ],
        out_specs=pl.BlockSpec((BLK, NCLS), lambda i: (i, 0)),
        out_shape=jax.ShapeDtypeStruct((N, NCLS), _F32),
    )(s0, s1, y2, deg, b2)


def kernel(x, edge_index, W1, b1, W2, b2):
    ei = edge_index.astype(jnp.int32)
    rowp = jnp.concatenate([ei[0], jnp.zeros((EP - E,), jnp.int32)])
    colp = jnp.concatenate([ei[1], jnp.full((EP - E,), DUMP, jnp.int32)])
    row3 = rowp.reshape(NW, NCH, CHUNK)
    col3 = colp.reshape(NW, NCH, CHUNK)
    row2 = rowp.reshape(NW * NCH, CHUNK)
    col2 = colp.reshape(NW * NCH, CHUNK)
    b1r = b1.reshape(1, HID)
    b2r = b2.reshape(1, NCLS)
    w2p = jnp.concatenate([W2, jnp.zeros((HID, HID - NCLS), _F32)], axis=1)

    hist = _deg_kernel(col3)
    xw = _matmul1(x, W1)
    deg = (hist.reshape(NW, NP).sum(axis=0)[:N] + 1.0).reshape(N, 1)
    y1 = _scale(xw, deg)
    s1 = _agg_kernel(y1, row2, col2)
    y2 = _mid(s1[:N], s1[NP:NP + N], y1, deg, b1r, w2p)
    s2 = _agg_kernel(y2, row2, col2)
    return _fin(s2[:N], s2[NP:NP + N], y2, deg, b2r)
